# baseline (speedup 1.0000x reference)
import jax
import jax.numpy as jnp
from jax import lax
from jax.experimental import pallas as pl
from jax.experimental.pallas import tpu as pltpu
from jax.experimental.pallas import tpu_sc as plsc

NH = 16
S = 32
N_TOK = S * S


def _sc_body(rh_hbm, rw_hbm, out_hbm, rh_v, rw_v, ch_v, cw_v,
             buf0, buf1, sem0, sem1):
    n = lax.axis_index("s")
    half = lax.axis_index("c")

    pltpu.sync_copy(rh_hbm, rh_v)
    pltpu.sync_copy(rw_hbm, rw_v)

    iota = lax.iota(jnp.int32, 16)
    n_vec = jnp.full((16,), n, jnp.int32)

    for j in range(4):
        rvec = jnp.maximum((2 * S - 2) - (iota + 16 * j), 0)
        ch_v[pl.ds(16 * j, 16)] = plsc.load_gather(rh_v, [rvec, n_vec])
        cw_v[pl.ds(16 * j, 16)] = plsc.load_gather(rw_v, [rvec, n_vec])

    def compute_chunk(h1, buf):
        a_vecs = [plsc.load_gather(
                      ch_v, [jnp.full((16,), 31 - h1 + h2, jnp.int32)])
                  for h2 in range(S)]

        def w1_body(w1, carry):
            b0 = plsc.load_gather(cw_v, [(31 - w1) + iota])
            b1 = plsc.load_gather(cw_v, [(47 - w1) + iota])
            for h2 in range(S):
                buf[w1, pl.ds(32 * h2, 16)] = b0 + a_vecs[h2]
                buf[w1, pl.ds(32 * h2 + 16, 16)] = b1 + a_vecs[h2]
            return carry

        lax.fori_loop(0, S, w1_body, 0)

    def cc_body(cc, carry):
        h1a = 16 * half + 2 * cc
        h1b = h1a + 1

        @pl.when(cc > 0)
        def _():
            pltpu.make_async_copy(
                buf0, out_hbm.at[n, pl.ds(32 * h1a, S), :], sem0).wait()

        compute_chunk(h1a, buf0)
        pltpu.async_copy(buf0, out_hbm.at[n, pl.ds(32 * h1a, S), :], sem0)

        @pl.when(cc > 0)
        def _():
            pltpu.make_async_copy(
                buf1, out_hbm.at[n, pl.ds(32 * h1b, S), :], sem1).wait()

        compute_chunk(h1b, buf1)
        pltpu.async_copy(buf1, out_hbm.at[n, pl.ds(32 * h1b, S), :], sem1)
        return carry

    lax.fori_loop(0, 8, cc_body, 0)

    tail = 16 * half + 14
    pltpu.make_async_copy(
        buf0, out_hbm.at[n, pl.ds(32 * tail, S), :], sem0).wait()
    pltpu.make_async_copy(
        buf1, out_hbm.at[n, pl.ds(32 * (tail + 1), S), :], sem1).wait()


@jax.jit
def _bias_sc(rh, rw):
    mesh = plsc.VectorSubcoreMesh(core_axis_name="c", subcore_axis_name="s")
    return pl.kernel(
        _sc_body,
        mesh=mesh,
        out_type=jax.ShapeDtypeStruct((NH, N_TOK, N_TOK), jnp.float32),
        scratch_types=[
            pltpu.VMEM((2 * S - 1, NH), jnp.float32),
            pltpu.VMEM((2 * S - 1, NH), jnp.float32),
            pltpu.VMEM((2 * S,), jnp.float32),
            pltpu.VMEM((2 * S,), jnp.float32),
            pltpu.VMEM((S, N_TOK), jnp.float32),
            pltpu.VMEM((S, N_TOK), jnp.float32),
            pltpu.SemaphoreType.DMA,
            pltpu.SemaphoreType.DMA,
        ],
        compiler_params=pltpu.CompilerParams(needs_layout_passes=False),
    )(rh, rw)


def kernel(rel_h, rel_w, H, W):
    return _bias_sc(rel_h, rel_w)


# speedup vs baseline: 1.0679x; 1.0679x over previous
import jax
import jax.numpy as jnp
from jax import lax
from jax.experimental import pallas as pl
from jax.experimental.pallas import tpu as pltpu
from jax.experimental.pallas import tpu_sc as plsc

NH = 16
S = 32
N_TOK = S * S


def _sc_body(fh_hbm, fw_hbm, out_hbm, fh_v, fw_v, buf0, buf1, sem0, sem1):
    n = lax.axis_index("s")
    half = lax.axis_index("c")

    pltpu.sync_copy(fh_hbm.at[n], fh_v)
    pltpu.sync_copy(fw_hbm.at[n], fw_v)

    iota = lax.iota(jnp.int32, 16)

    def compute_chunk(h1, buf):
        a_base = 31 - h1
        a_vecs = [plsc.load_gather(fh_v, [jnp.full((16,), a_base + h2,
                                                   jnp.int32)])
                  for h2 in range(S)]

        def w1_body(w1, carry):
            idx = (31 - w1) + iota
            b0 = plsc.load_gather(fw_v, [idx])
            b1 = plsc.load_gather(fw_v, [idx + 16])
            for h2 in range(S):
                buf[w1, pl.ds(32 * h2, 16)] = b0 + a_vecs[h2]
                buf[w1, pl.ds(32 * h2 + 16, 16)] = b1 + a_vecs[h2]
            return carry

        lax.fori_loop(0, S, w1_body, 0)

    def cc_body(cc, carry):
        h1a = 16 * half + 2 * cc
        h1b = h1a + 1

        @pl.when(cc > 0)
        def _():
            pltpu.make_async_copy(
                buf0, out_hbm.at[n, pl.ds(32 * h1a, S), :], sem0).wait()

        compute_chunk(h1a, buf0)
        pltpu.async_copy(buf0, out_hbm.at[n, pl.ds(32 * h1a, S), :], sem0)

        @pl.when(cc > 0)
        def _():
            pltpu.make_async_copy(
                buf1, out_hbm.at[n, pl.ds(32 * h1b, S), :], sem1).wait()

        compute_chunk(h1b, buf1)
        pltpu.async_copy(buf1, out_hbm.at[n, pl.ds(32 * h1b, S), :], sem1)
        return carry

    lax.fori_loop(0, 8, cc_body, 0)

    tail = 16 * half + 14
    pltpu.make_async_copy(
        buf0, out_hbm.at[n, pl.ds(32 * tail, S), :], sem0).wait()
    pltpu.make_async_copy(
        buf1, out_hbm.at[n, pl.ds(32 * (tail + 1), S), :], sem1).wait()


@jax.jit
def _bias_sc(fh, fw):
    mesh = plsc.VectorSubcoreMesh(core_axis_name="c", subcore_axis_name="s")
    return pl.kernel(
        _sc_body,
        mesh=mesh,
        out_type=jax.ShapeDtypeStruct((NH, N_TOK, N_TOK), jnp.float32),
        scratch_types=[
            pltpu.VMEM((2 * S,), jnp.float32),
            pltpu.VMEM((2 * S,), jnp.float32),
            pltpu.VMEM((S, N_TOK), jnp.float32),
            pltpu.VMEM((S, N_TOK), jnp.float32),
            pltpu.SemaphoreType.DMA,
            pltpu.SemaphoreType.DMA,
        ],
        compiler_params=pltpu.CompilerParams(needs_layout_passes=False),
    )(fh, fw)


def kernel(rel_h, rel_w, H, W):
    fh = jnp.pad(jnp.flip(rel_h, axis=0).T, ((0, 0), (0, 1)))
    fw = jnp.pad(jnp.flip(rel_w, axis=0).T, ((0, 0), (0, 1)))
    return _bias_sc(fh, fw)


# speedup vs baseline: 1.0683x; 1.0004x over previous
import jax
import jax.numpy as jnp
from jax import lax
from jax.experimental import pallas as pl
from jax.experimental.pallas import tpu as pltpu
from jax.experimental.pallas import tpu_sc as plsc

NH = 16
S = 32
N_TOK = S * S


def _sc_body(g_hbm, out_hbm, fh_v, fw_v, buf0, buf1, sem0, sem1):
    n = lax.axis_index("s")
    half = lax.axis_index("c")

    pltpu.sync_copy(g_hbm.at[n], fh_v)
    pltpu.sync_copy(g_hbm.at[NH + n], fw_v)

    iota = lax.iota(jnp.int32, 16)

    def compute_chunk(h1, buf):
        a_base = 31 - h1
        a_vecs = [plsc.load_gather(fh_v, [jnp.full((16,), a_base + h2,
                                                   jnp.int32)])
                  for h2 in range(S)]

        def w1_body(w1, carry):
            idx = (31 - w1) + iota
            b0 = plsc.load_gather(fw_v, [idx])
            b1 = plsc.load_gather(fw_v, [idx + 16])
            for h2 in range(S):
                buf[w1, pl.ds(32 * h2, 16)] = b0 + a_vecs[h2]
                buf[w1, pl.ds(32 * h2 + 16, 16)] = b1 + a_vecs[h2]
            return carry

        lax.fori_loop(0, S, w1_body, 0)

    def cc_body(cc, carry):
        h1a = 16 * half + 2 * cc
        h1b = h1a + 1

        @pl.when(cc > 0)
        def _():
            pltpu.make_async_copy(
                buf0, out_hbm.at[n, pl.ds(32 * h1a, S), :], sem0).wait()

        compute_chunk(h1a, buf0)
        pltpu.async_copy(buf0, out_hbm.at[n, pl.ds(32 * h1a, S), :], sem0)

        @pl.when(cc > 0)
        def _():
            pltpu.make_async_copy(
                buf1, out_hbm.at[n, pl.ds(32 * h1b, S), :], sem1).wait()

        compute_chunk(h1b, buf1)
        pltpu.async_copy(buf1, out_hbm.at[n, pl.ds(32 * h1b, S), :], sem1)
        return carry

    lax.fori_loop(0, 8, cc_body, 0)

    tail = 16 * half + 14
    pltpu.make_async_copy(
        buf0, out_hbm.at[n, pl.ds(32 * tail, S), :], sem0).wait()
    pltpu.make_async_copy(
        buf1, out_hbm.at[n, pl.ds(32 * (tail + 1), S), :], sem1).wait()


@jax.jit
def _bias_sc(g):
    mesh = plsc.VectorSubcoreMesh(core_axis_name="c", subcore_axis_name="s")
    return pl.kernel(
        _sc_body,
        mesh=mesh,
        out_type=jax.ShapeDtypeStruct((NH, N_TOK, N_TOK), jnp.float32),
        scratch_types=[
            pltpu.VMEM((2 * S,), jnp.float32),
            pltpu.VMEM((2 * S,), jnp.float32),
            pltpu.VMEM((S, N_TOK), jnp.float32),
            pltpu.VMEM((S, N_TOK), jnp.float32),
            pltpu.SemaphoreType.DMA,
            pltpu.SemaphoreType.DMA,
        ],
        compiler_params=pltpu.CompilerParams(needs_layout_passes=False),
    )(g)


def kernel(rel_h, rel_w, H, W):
    g = jnp.pad(jnp.flip(jnp.concatenate([rel_h, rel_w], axis=1),
                         axis=0).T, ((0, 0), (0, 1)))
    return _bias_sc(g)


# speedup vs baseline: 1.0820x; 1.0127x over previous
import jax
import jax.numpy as jnp
from jax import lax
from jax.experimental import pallas as pl
from jax.experimental.pallas import tpu as pltpu
from jax.experimental.pallas import tpu_sc as plsc

NH = 16
S = 32
N_TOK = S * S


def _sc_body(g_hbm, out_hbm, t_v, buf0, buf1, sem0, sem1):
    n = lax.axis_index("s")
    half = lax.axis_index("c")

    pltpu.sync_copy(g_hbm.at[n], t_v)

    iota = lax.iota(jnp.int32, 16)

    def compute_chunk(h1, buf):
        a_base = 31 - h1
        a_vecs = [plsc.load_gather(t_v, [jnp.full((16,), a_base + h2,
                                                  jnp.int32)])
                  for h2 in range(S)]

        def w1_body(w1, carry):
            idx = (95 - w1) + iota
            b0 = plsc.load_gather(t_v, [idx])
            b1 = plsc.load_gather(t_v, [idx + 16])
            for h2 in range(S):
                buf[w1, pl.ds(32 * h2, 16)] = b0 + a_vecs[h2]
                buf[w1, pl.ds(32 * h2 + 16, 16)] = b1 + a_vecs[h2]
            return carry

        lax.fori_loop(0, S, w1_body, 0)

    def cc_body(cc, carry):
        h1a = 16 * half + 2 * cc
        h1b = h1a + 1

        @pl.when(cc > 0)
        def _():
            pltpu.make_async_copy(
                buf0, out_hbm.at[n, pl.ds(32 * h1a, S), :], sem0).wait()

        compute_chunk(h1a, buf0)
        pltpu.async_copy(buf0, out_hbm.at[n, pl.ds(32 * h1a, S), :], sem0)

        @pl.when(cc > 0)
        def _():
            pltpu.make_async_copy(
                buf1, out_hbm.at[n, pl.ds(32 * h1b, S), :], sem1).wait()

        compute_chunk(h1b, buf1)
        pltpu.async_copy(buf1, out_hbm.at[n, pl.ds(32 * h1b, S), :], sem1)
        return carry

    lax.fori_loop(0, 8, cc_body, 0)

    tail = 16 * half + 14
    pltpu.make_async_copy(
        buf0, out_hbm.at[n, pl.ds(32 * tail, S), :], sem0).wait()
    pltpu.make_async_copy(
        buf1, out_hbm.at[n, pl.ds(32 * (tail + 1), S), :], sem1).wait()


@jax.jit
def _bias_sc(g):
    mesh = plsc.VectorSubcoreMesh(core_axis_name="c", subcore_axis_name="s")
    return pl.kernel(
        _sc_body,
        mesh=mesh,
        out_type=jax.ShapeDtypeStruct((NH, N_TOK, N_TOK), jnp.float32),
        scratch_types=[
            pltpu.VMEM((4 * S,), jnp.float32),
            pltpu.VMEM((S, N_TOK), jnp.float32),
            pltpu.VMEM((S, N_TOK), jnp.float32),
            pltpu.SemaphoreType.DMA,
            pltpu.SemaphoreType.DMA,
        ],
        compiler_params=pltpu.CompilerParams(needs_layout_passes=False),
    )(g)


def kernel(rel_h, rel_w, H, W):
    fh = jnp.pad(jnp.flip(rel_h, axis=0).T, ((0, 0), (0, 1)))
    fw = jnp.pad(jnp.flip(rel_w, axis=0).T, ((0, 0), (0, 1)))
    return _bias_sc(jnp.concatenate([fh, fw], axis=1))


# speedup vs baseline: 1.0923x; 1.0096x over previous
import jax
import jax.numpy as jnp
from jax import lax
from jax.experimental import pallas as pl
from jax.experimental.pallas import tpu as pltpu
from jax.experimental.pallas import tpu_sc as plsc

NH = 16
S = 32
N_TOK = S * S


def _sc_body(g_hbm, out_hbm, t_v, buf0, buf1, sem0, sem1):
    n = lax.axis_index("s")
    half = lax.axis_index("c")

    pltpu.sync_copy(g_hbm.at[n], t_v)

    iota = lax.iota(jnp.int32, 16)

    H16 = S // 2

    def compute_chunk(h1, buf, sem):
        a_base = 31 - h1
        a_vecs = [plsc.load_gather(t_v, [jnp.full((16,), a_base + h2,
                                                  jnp.int32)])
                  for h2 in range(S)]

        def w1_body(w1, carry):
            idx = (95 - w1) + iota
            b0 = plsc.load_gather(t_v, [idx])
            b1 = plsc.load_gather(t_v, [idx + 16])
            for h2 in range(S):
                buf[w1, pl.ds(32 * h2, 16)] = b0 + a_vecs[h2]
                buf[w1, pl.ds(32 * h2 + 16, 16)] = b1 + a_vecs[h2]

            @pl.when(w1 == H16 - 1)
            def _():
                pltpu.async_copy(
                    buf.at[pl.ds(0, H16)],
                    out_hbm.at[n, pl.ds(32 * h1, H16), :], sem)

            return carry

        lax.fori_loop(0, S, w1_body, 0)
        pltpu.async_copy(buf.at[pl.ds(H16, H16)],
                         out_hbm.at[n, pl.ds(32 * h1 + H16, H16), :], sem)

    def wait_chunk(h1, buf, sem):
        pltpu.make_async_copy(
            buf, out_hbm.at[n, pl.ds(32 * h1, S), :], sem).wait()

    def cc_body(cc, carry):
        h1a = 16 * half + 2 * cc
        h1b = h1a + 1

        @pl.when(cc > 0)
        def _():
            wait_chunk(h1a - 2, buf0, sem0)

        compute_chunk(h1a, buf0, sem0)

        @pl.when(cc > 0)
        def _():
            wait_chunk(h1b - 2, buf1, sem1)

        compute_chunk(h1b, buf1, sem1)
        return carry

    lax.fori_loop(0, 8, cc_body, 0)

    tail = 16 * half + 14
    wait_chunk(tail, buf0, sem0)
    wait_chunk(tail + 1, buf1, sem1)


@jax.jit
def _bias_sc(g):
    mesh = plsc.VectorSubcoreMesh(core_axis_name="c", subcore_axis_name="s")
    return pl.kernel(
        _sc_body,
        mesh=mesh,
        out_type=jax.ShapeDtypeStruct((NH, N_TOK, N_TOK), jnp.float32),
        scratch_types=[
            pltpu.VMEM((4 * S,), jnp.float32),
            pltpu.VMEM((S, N_TOK), jnp.float32),
            pltpu.VMEM((S, N_TOK), jnp.float32),
            pltpu.SemaphoreType.DMA,
            pltpu.SemaphoreType.DMA,
        ],
        compiler_params=pltpu.CompilerParams(needs_layout_passes=False),
    )(g)


def kernel(rel_h, rel_w, H, W):
    fh = jnp.pad(jnp.flip(rel_h, axis=0).T, ((0, 0), (0, 1)))
    fw = jnp.pad(jnp.flip(rel_w, axis=0).T, ((0, 0), (0, 1)))
    return _bias_sc(jnp.concatenate([fh, fw], axis=1))
